# baseline (speedup 1.0000x reference)
import functools

import jax
import jax.numpy as jnp
from jax import lax
from jax.experimental import pallas as pl
from jax.experimental.pallas import tpu as pltpu
from jax.experimental.pallas import tpu_sc as plsc

MAXD = 512
CMAX = 3072
TM = 256
NW = 32
RPW = CMAX // NW
LANES = 16


def _sc_mesh():
    return plsc.VectorSubcoreMesh(core_axis_name="c", subcore_axis_name="s")


def _wid():
    return lax.axis_index("s") * 2 + lax.axis_index("c")


N_SCHED = 65536
NWRD = N_SCHED // 32
KSCH = 4
NW1 = LANES
SPW = N_SCHED // NW1
WPW = NWRD // NW1


@functools.lru_cache(maxsize=None)
def _build_sched_kernel():
    @functools.partial(
        pl.kernel,
        out_type=(
            jax.ShapeDtypeStruct((NWRD,), jnp.int32),
            jax.ShapeDtypeStruct((N_SCHED,), jnp.int32),
            jax.ShapeDtypeStruct((NW1, KSCH * 3, LANES), jnp.int32),
        ),
        mesh=plsc.VectorSubcoreMesh(
            core_axis_name="c", subcore_axis_name="s", num_cores=1),
        scratch_types=[
            pltpu.VMEM((NWRD,), jnp.int32),
            pltpu.VMEM((SPW,), jnp.int32),
            pltpu.VMEM((SPW,), jnp.int32),
            pltpu.VMEM((SPW,), jnp.int32),
            pltpu.VMEM((SPW,), jnp.int32),
            pltpu.VMEM((WPW,), jnp.int32),
            pltpu.VMEM((LANES,), jnp.int32),
            pltpu.VMEM((KSCH * 3, LANES), jnp.int32),
            pltpu.SemaphoreType.DMA,
        ],
        compiler_params=pltpu.CompilerParams(needs_layout_passes=False),
        name="fen_sched",
    )
    def sched_kernel(t_hbm, xe_hbm, ye_hbm, done_hbm,
                     lev_hbm, done_out, lev_out, cnt_hbm,
                     dbits_v, xe_v, ye_v, lev_v, rdy_v,
                     dnew_v, t_v, acc_v, sem):
        wid = lax.axis_index("s")
        base = wid * SPW
        wbase = wid * WPW
        c0 = pltpu.async_copy(done_hbm, dbits_v, sem)
        c1 = pltpu.async_copy(xe_hbm.at[pl.ds(base, SPW)], xe_v, sem)
        c2 = pltpu.async_copy(ye_hbm.at[pl.ds(base, SPW)], ye_v, sem)
        c3 = pltpu.async_copy(lev_hbm.at[pl.ds(base, SPW)], lev_v, sem)
        c4 = pltpu.async_copy(t_hbm, t_v, sem)
        c0.wait(); c1.wait(); c2.wait(); c3.wait(); c4.wait()
        iota = lax.iota(jnp.int32, LANES)
        one = jnp.ones((LANES,), jnp.int32)
        five = jnp.full((LANES,), 5, jnp.int32)
        m31 = jnp.full((LANES,), 31, jnp.int32)
        m16 = jnp.full((LANES,), 16, jnp.int32)
        m17 = jnp.full((LANES,), 17, jnp.int32)
        mlow = jnp.full((LANES,), 0xFFFF, jnp.int32)

        def bit_of(idx):
            w = plsc.load_gather(dbits_v, [lax.shift_right_logical(idx, five)])
            return lax.shift_right_logical(w, idx & m31) & one

        for k in range(KSCH):
            acc_v[3 * k + 0, :] = jnp.zeros((LANES,), jnp.int32)
            acc_v[3 * k + 1, :] = jnp.zeros((LANES,), jnp.int32)
            acc_v[3 * k + 2, :] = jnp.zeros((LANES,), jnp.int32)
            tk = t_v[...] + k

            @pl.loop(0, SPW // LANES)
            def _(i, k=k, tk=tk):
                sl = pl.ds(i * LANES, LANES)
                xa = xe_v[sl]
                inp = lax.shift_right_logical(xa, m16) & one
                isand = lax.shift_right_logical(xa, m17) & one
                d = bit_of(base + i * LANES + iota)
                ready = (1 - d) & (inp | (bit_of(xa & mlow) & bit_of(ye_v[sl])))
                rdy_v[sl] = ready
                lev_v[sl] = jnp.where(ready == 1, tk, lev_v[sl])
                acc_v[3 * k + 0, :] = acc_v[3 * k + 0, :] + ready
                acc_v[3 * k + 1, :] = acc_v[3 * k + 1, :] + (
                    ready & (1 - isand) & (1 - inp))
                acc_v[3 * k + 2, :] = acc_v[3 * k + 2, :] + (ready & isand)

            @pl.loop(0, WPW // LANES)
            def _(wc):
                words = jnp.zeros((LANES,), jnp.int32)
                for b in range(32):
                    bits = plsc.load_gather(rdy_v, [wc * 512 + iota * 32 + b])
                    words = words | lax.shift_left(
                        bits, jnp.full((LANES,), b, jnp.int32))
                old = dbits_v[pl.ds(wbase + wc * LANES, LANES)]
                dnew_v[pl.ds(wc * LANES, LANES)] = old | words

            pltpu.sync_copy(dnew_v, done_out.at[pl.ds(wbase, WPW)])
            if k < KSCH - 1:
                plsc.subcore_barrier()
                pltpu.sync_copy(done_out, dbits_v)
                plsc.subcore_barrier()

        pltpu.sync_copy(lev_v, lev_out.at[pl.ds(base, SPW)])
        pltpu.sync_copy(acc_v, cnt_hbm.at[wid])

    return sched_kernel


@functools.lru_cache(maxsize=None)
def _build_level_kernel(n_pad, e, cmax):
    rpw = cmax // NW

    @functools.partial(
        pl.kernel,
        out_type=(
            jax.ShapeDtypeStruct((cmax, e), jnp.float32),
            jax.ShapeDtypeStruct((cmax, e), jnp.float32),
        ),
        mesh=_sc_mesh(),
        scratch_types=[
            pltpu.VMEM((rpw,), jnp.int32),
            pltpu.VMEM((rpw,), jnp.int32),
            pltpu.VMEM((rpw,), jnp.int32),
            pltpu.VMEM((rpw,), jnp.int32),
            pltpu.VMEM((rpw,), jnp.int32),
            pltpu.VMEM((rpw, e), jnp.float32),
            pltpu.VMEM((rpw, e), jnp.float32),
            pltpu.VMEM((rpw, e), jnp.float32),
            pltpu.SemaphoreType.DMA,
        ],
        name=f"fen_level_{cmax}",
    )
    def level_kernel(idn_hbm, ida_hbm, xe_hbm, ye_hbm, embd_ref,
                     hx_hbm, hy_hbm,
                     idn_v, ida_v, xn_v, xs_v, ys_v, rn_v, hx_v, hy_v, sem):
        base = _wid() * rpw
        pltpu.sync_copy(idn_hbm.at[pl.ds(base, rpw)], idn_v)
        pltpu.sync_copy(ida_hbm.at[pl.ds(base, rpw)], ida_v)
        c1 = pltpu.async_copy(xe_hbm.at[idn_v], xn_v, sem)
        c2 = pltpu.async_copy(xe_hbm.at[ida_v], xs_v, sem)
        c3 = pltpu.async_copy(ye_hbm.at[ida_v], ys_v, sem)
        c1.wait(); c2.wait(); c3.wait()
        c4 = pltpu.async_copy(embd_ref.at[xn_v], rn_v, sem)
        c5 = pltpu.async_copy(embd_ref.at[xs_v], hx_v, sem)
        c6 = pltpu.async_copy(embd_ref.at[ys_v], hy_v, sem)
        c4.wait(); c5.wait(); c6.wait()

        @pl.loop(0, rpw)
        def _(i):
            for j in range(e // LANES):
                sl = (i, pl.ds(j * LANES, LANES))
                rn_v[sl] = -rn_v[sl]

        c7 = pltpu.async_copy(rn_v, embd_ref.at[idn_v], sem)
        pltpu.sync_copy(hx_v, hx_hbm.at[pl.ds(base, rpw)])
        pltpu.sync_copy(hy_v, hy_hbm.at[pl.ds(base, rpw)])
        c7.wait()

    return level_kernel


@functools.lru_cache(maxsize=None)
def _build_level_merged_kernel(n_pad, e, cmax):
    nw1 = LANES
    rpw = cmax // nw1
    mesh = plsc.VectorSubcoreMesh(
        core_axis_name="c", subcore_axis_name="s", num_cores=1)

    @functools.partial(
        pl.kernel,
        out_type=(
            jax.ShapeDtypeStruct((cmax, e), jnp.float32),
            jax.ShapeDtypeStruct((cmax, e), jnp.float32),
        ),
        mesh=mesh,
        scratch_types=[
            pltpu.VMEM((rpw,), jnp.int32),
            pltpu.VMEM((rpw, e), jnp.float32),
            pltpu.VMEM((rpw,), jnp.int32),
            pltpu.VMEM((rpw,), jnp.int32),
            pltpu.VMEM((rpw,), jnp.int32),
            pltpu.VMEM((rpw,), jnp.int32),
            pltpu.VMEM((rpw,), jnp.int32),
            pltpu.VMEM((rpw, e), jnp.float32),
            pltpu.VMEM((rpw, e), jnp.float32),
            pltpu.VMEM((rpw, e), jnp.float32),
            pltpu.SemaphoreType.DMA,
        ],
        name=f"fen_level_m{cmax}",
    )
    def level_merged(pid_hbm, prow_hbm, idn_hbm, ida_hbm, xe_hbm, ye_hbm,
                     embd_ref, hx_hbm, hy_hbm,
                     pid_v, prow_v, idn_v, ida_v, xn_v, xs_v, ys_v,
                     rn_v, hx_v, hy_v, sem):
        base = lax.axis_index("s") * rpw
        a1 = pltpu.async_copy(pid_hbm.at[pl.ds(base, rpw)], pid_v, sem)
        a2 = pltpu.async_copy(prow_hbm.at[pl.ds(base, rpw)], prow_v, sem)
        a3 = pltpu.async_copy(idn_hbm.at[pl.ds(base, rpw)], idn_v, sem)
        a4 = pltpu.async_copy(ida_hbm.at[pl.ds(base, rpw)], ida_v, sem)
        a1.wait(); a2.wait()
        pltpu.async_copy(prow_v, embd_ref.at[pid_v], sem).wait()
        a3.wait(); a4.wait()
        c1 = pltpu.async_copy(xe_hbm.at[idn_v], xn_v, sem)
        c2 = pltpu.async_copy(xe_hbm.at[ida_v], xs_v, sem)
        c3 = pltpu.async_copy(ye_hbm.at[ida_v], ys_v, sem)
        plsc.subcore_barrier()
        c1.wait(); c2.wait(); c3.wait()
        c4 = pltpu.async_copy(embd_ref.at[xn_v], rn_v, sem)
        c5 = pltpu.async_copy(embd_ref.at[xs_v], hx_v, sem)
        c6 = pltpu.async_copy(embd_ref.at[ys_v], hy_v, sem)
        c4.wait(); c5.wait(); c6.wait()

        @pl.loop(0, rpw)
        def _(i):
            for j in range(e // LANES):
                sl = (i, pl.ds(j * LANES, LANES))
                rn_v[sl] = -rn_v[sl]

        c7 = pltpu.async_copy(rn_v, embd_ref.at[idn_v], sem)
        pltpu.sync_copy(hx_v, hx_hbm.at[pl.ds(base, rpw)])
        pltpu.sync_copy(hy_v, hy_hbm.at[pl.ds(base, rpw)])
        c7.wait()

    return level_merged


@functools.lru_cache(maxsize=None)
def _build_scatter_kernel(n_pad, e, cmax):
    rpw = cmax // NW

    @functools.partial(
        pl.kernel,
        out_type=(),
        mesh=_sc_mesh(),
        scratch_types=[
            pltpu.VMEM((rpw,), jnp.int32),
            pltpu.VMEM((rpw, e), jnp.float32),
            pltpu.SemaphoreType.DMA,
        ],
        name=f"fen_scatter_{cmax}",
    )
    def scatter_kernel(ids_hbm, rows_hbm, embd_ref, ids_v, rows_v, sem):
        base = _wid() * rpw
        pltpu.sync_copy(ids_hbm.at[pl.ds(base, rpw)], ids_v)
        pltpu.sync_copy(rows_hbm.at[pl.ds(base, rpw)], rows_v)
        pltpu.async_copy(rows_v, embd_ref.at[ids_v], sem).wait()

    return scatter_kernel


CMAX_S = 768


def _mlp_body(cnt_ref, hx_ref, hy_ref, w0x_ref, w0y_ref, b0_ref, w1_ref,
              b1_ref, g_ref, bb_ref, out_ref):
    t = pl.program_id(0)

    @pl.when(t * TM < cnt_ref[0])
    def _():
        hx = hx_ref[...]
        hy = hy_ref[...]
        z = lax.dot_general(hx, w0x_ref[...], (((1,), (1,)), ((), ())),
                            preferred_element_type=jnp.float32)
        z += lax.dot_general(hy, w0y_ref[...], (((1,), (1,)), ((), ())),
                             preferred_element_type=jnp.float32)
        z = jnp.maximum(z + b0_ref[...], 0.0)
        o = lax.dot_general(z, w1_ref[...], (((1,), (1,)), ((), ())),
                            preferred_element_type=jnp.float32)
        o = o + b1_ref[...]
        mu = jnp.mean(o, axis=-1, keepdims=True)
        var = jnp.mean((o - mu) ** 2, axis=-1, keepdims=True)
        out_ref[...] = (o - mu) * lax.rsqrt(var + 1e-5) * g_ref[...] + bb_ref[...]


@functools.lru_cache(maxsize=None)
def _build_mlp_kernel(e, h, cmax):
    grid = (cmax // TM,)
    return pl.pallas_call(
        _mlp_body,
        grid_spec=pltpu.PrefetchScalarGridSpec(
            num_scalar_prefetch=1,
            grid=grid,
            in_specs=[
                pl.BlockSpec((TM, e), lambda t, cnt: (t, 0)),
                pl.BlockSpec((TM, e), lambda t, cnt: (t, 0)),
                pl.BlockSpec((h, e), lambda t, cnt: (0, 0)),
                pl.BlockSpec((h, e), lambda t, cnt: (0, 0)),
                pl.BlockSpec((1, h), lambda t, cnt: (0, 0)),
                pl.BlockSpec((e, h), lambda t, cnt: (0, 0)),
                pl.BlockSpec((1, e), lambda t, cnt: (0, 0)),
                pl.BlockSpec((1, e), lambda t, cnt: (0, 0)),
                pl.BlockSpec((1, e), lambda t, cnt: (0, 0)),
            ],
            out_specs=pl.BlockSpec((TM, e), lambda t, cnt: (t, 0)),
        ),
        out_shape=jax.ShapeDtypeStruct((cmax, e), jnp.float32),
    )


def kernel(emb, W0, b0, W1, b1, ln_g, ln_b, nodes, x_edges, y_edges):
    n, e = emb.shape
    hdim = W0.shape[0]
    n_pad = n + CMAX
    is_input = nodes == 0
    n_inputs = jnp.sum(is_input)

    init = jnp.where(jnp.arange(n)[:, None] < n_inputs, emb,
                     jnp.zeros((n, e), emb.dtype))
    embd_ext = jnp.concatenate(
        [init, jnp.zeros((n_pad - n, e), emb.dtype)], axis=0)
    dummy_tail = jnp.arange(n, n_pad, dtype=jnp.int32)
    xe_ext = jnp.concatenate([x_edges.astype(jnp.int32), dummy_tail])
    ye_ext = jnp.concatenate([y_edges.astype(jnp.int32), dummy_tail])

    big = jnp.int32(0x3FFFFFFF)
    pad_sched = N_SCHED - n
    xe_sched = jnp.concatenate(
        [x_edges.astype(jnp.int32)
         | (is_input.astype(jnp.int32) << 16)
         | ((nodes == 1).astype(jnp.int32) << 17),
         jnp.full((pad_sched,), n, jnp.int32)])
    ye_sched = jnp.concatenate(
        [y_edges.astype(jnp.int32), jnp.full((pad_sched,), n, jnp.int32)])
    sched_k = _build_sched_kernel()

    def sched_cond(state):
        t, cnt, _, _, _ = state
        return cnt > 0

    def sched_body(state):
        t, _, done, lev, cnts = state
        t_arr = jnp.full((LANES,), t, jnp.int32)
        done, lev, counts = sched_k(t_arr, xe_sched, ye_sched, done, lev)
        rows = jnp.sum(counts, axis=(0, 2)).reshape(KSCH, 3)
        cnts = lax.dynamic_update_slice(
            cnts, rows, (jnp.minimum(t, MAXD - KSCH), 0))
        return t + KSCH, rows[KSCH - 1, 0], done, lev, cnts

    state0 = (jnp.int32(0), jnp.int32(1), jnp.zeros((NWRD,), jnp.int32),
              jnp.full((N_SCHED,), big, jnp.int32),
              jnp.zeros((MAXD, 3), jnp.int32))
    state0 = lax.fori_loop(0, 5, lambda i, s: sched_body(s), state0)
    _, _, _, lev_full, cnts = lax.while_loop(
        sched_cond, sched_body, state0)
    lev = lev_full[:n]
    depth_levels = jnp.sum((cnts[:, 0] > 0).astype(jnp.int32))

    key = jnp.where((lev > 0) & (lev < big),
                    lev * 2 + (nodes == 1).astype(jnp.int32),
                    jnp.int32(2 * MAXD + 2))
    key = jnp.minimum(key, 2 * MAXD + 2)
    packed = (key << 16) | jnp.arange(n, dtype=jnp.int32)
    order = lax.sort(packed) & jnp.int32(0xFFFF)
    offs = jnp.concatenate(
        [jnp.zeros((1,), jnp.int32), jnp.cumsum(cnts[:, 1:3].reshape(-1))])
    order_pad = jnp.concatenate(
        [order, jnp.full((CMAX,), n, dtype=jnp.int32)])

    level_kb = _build_level_kernel(n_pad, e, CMAX)
    scatter_kb = _build_scatter_kernel(n_pad, e, CMAX)
    mlp_kb = _build_mlp_kernel(e, hdim, CMAX)
    merged_ks = _build_level_merged_kernel(n_pad, e, CMAX_S)
    scatter_ks = _build_scatter_kernel(n_pad, e, CMAX_S)
    mlp_ks = _build_mlp_kernel(e, hdim, CMAX_S)

    w0x = W0[:, :e]
    w0y = W0[:, e:]
    b0r = b0.reshape(1, hdim)
    b1r = b1.reshape(1, e)
    gr = ln_g.reshape(1, e)
    br = ln_b.reshape(1, e)

    embd_ref = jax.new_ref(embd_ext)

    def make_ids(l, cmax):
        slot = jnp.arange(cmax, dtype=jnp.int32)
        dummy_ids = slot + n
        s0 = offs[2 * l]
        s1 = offs[2 * l + 1]
        s2 = offs[2 * l + 2]
        ids_not = lax.dynamic_slice(order_pad, (s0,), (cmax,))
        ids_not = jnp.where(slot < s1 - s0, ids_not, dummy_ids)
        cnt_and = s2 - s1
        ids_and = lax.dynamic_slice(order_pad, (s1,), (cmax,))
        ids_and = jnp.where(slot < cnt_and, ids_and, dummy_ids)
        return ids_not, ids_and, cnt_and

    def level_big(l):
        ids_not, ids_and, cnt_and = make_ids(l, CMAX)
        hx, hy = level_kb(ids_not, ids_and, xe_ext, ye_ext, embd_ref)
        out = mlp_kb(cnt_and.reshape(1), hx, hy, w0x, w0y, b0r, W1, b1r,
                     gr, br)
        scatter_kb(ids_and, out, embd_ref)

    level_big(jnp.int32(1))
    level_big(jnp.int32(2))

    def level_body(l, carry):
        pids, pout = carry
        ids_not, ids_and, cnt_and = make_ids(l, CMAX_S)
        hx, hy = merged_ks(pids, pout, ids_not, ids_and, xe_ext, ye_ext,
                           embd_ref)
        out = mlp_ks(cnt_and.reshape(1), hx, hy, w0x, w0y, b0r, W1, b1r,
                     gr, br)
        return ids_and, out

    dummy_s = jnp.arange(CMAX_S, dtype=jnp.int32) + n
    pids, pout = lax.fori_loop(
        3, jnp.minimum(depth_levels, MAXD), level_body,
        (dummy_s, jnp.zeros((CMAX_S, e), jnp.float32)))
    scatter_ks(pids, pout, embd_ref)
    return embd_ref[...][:n]


# speedup vs baseline: 1.0155x; 1.0155x over previous
import functools

import jax
import jax.numpy as jnp
from jax import lax
from jax.experimental import pallas as pl
from jax.experimental.pallas import tpu as pltpu
from jax.experimental.pallas import tpu_sc as plsc

MAXD = 512
CMAX = 3072
TM = 256
NW = 32
RPW = CMAX // NW
LANES = 16


def _sc_mesh():
    return plsc.VectorSubcoreMesh(core_axis_name="c", subcore_axis_name="s")


def _wid():
    return lax.axis_index("s") * 2 + lax.axis_index("c")


N_SCHED = 65536
NWRD = N_SCHED // 32
KSCH = 3
NW1 = LANES
SPW = N_SCHED // NW1
WPW = NWRD // NW1


@functools.lru_cache(maxsize=None)
def _build_sched_kernel():
    @functools.partial(
        pl.kernel,
        out_type=(
            jax.ShapeDtypeStruct((NWRD,), jnp.int32),
            jax.ShapeDtypeStruct((N_SCHED,), jnp.int32),
            jax.ShapeDtypeStruct((NW1, KSCH * 3, LANES), jnp.int32),
        ),
        mesh=plsc.VectorSubcoreMesh(
            core_axis_name="c", subcore_axis_name="s", num_cores=1),
        scratch_types=[
            pltpu.VMEM((NWRD,), jnp.int32),
            pltpu.VMEM((SPW,), jnp.int32),
            pltpu.VMEM((SPW,), jnp.int32),
            pltpu.VMEM((SPW,), jnp.int32),
            pltpu.VMEM((SPW,), jnp.int32),
            pltpu.VMEM((WPW,), jnp.int32),
            pltpu.VMEM((LANES,), jnp.int32),
            pltpu.VMEM((KSCH * 3, LANES), jnp.int32),
            pltpu.SemaphoreType.DMA,
        ],
        compiler_params=pltpu.CompilerParams(needs_layout_passes=False),
        name="fen_sched",
    )
    def sched_kernel(t_hbm, xe_hbm, ye_hbm, done_hbm,
                     lev_hbm, done_out, lev_out, cnt_hbm,
                     dbits_v, xe_v, ye_v, lev_v, rdy_v,
                     dnew_v, t_v, acc_v, sem):
        wid = lax.axis_index("s")
        base = wid * SPW
        wbase = wid * WPW
        c0 = pltpu.async_copy(done_hbm, dbits_v, sem)
        c1 = pltpu.async_copy(xe_hbm.at[pl.ds(base, SPW)], xe_v, sem)
        c2 = pltpu.async_copy(ye_hbm.at[pl.ds(base, SPW)], ye_v, sem)
        c3 = pltpu.async_copy(lev_hbm.at[pl.ds(base, SPW)], lev_v, sem)
        c4 = pltpu.async_copy(t_hbm, t_v, sem)
        c0.wait(); c1.wait(); c2.wait(); c3.wait(); c4.wait()
        iota = lax.iota(jnp.int32, LANES)
        one = jnp.ones((LANES,), jnp.int32)
        five = jnp.full((LANES,), 5, jnp.int32)
        m31 = jnp.full((LANES,), 31, jnp.int32)
        m16 = jnp.full((LANES,), 16, jnp.int32)
        m17 = jnp.full((LANES,), 17, jnp.int32)
        mlow = jnp.full((LANES,), 0xFFFF, jnp.int32)

        def bit_of(idx):
            w = plsc.load_gather(dbits_v, [lax.shift_right_logical(idx, five)])
            return lax.shift_right_logical(w, idx & m31) & one

        for k in range(KSCH):
            acc_v[3 * k + 0, :] = jnp.zeros((LANES,), jnp.int32)
            acc_v[3 * k + 1, :] = jnp.zeros((LANES,), jnp.int32)
            acc_v[3 * k + 2, :] = jnp.zeros((LANES,), jnp.int32)
            tk = t_v[...] + k

            @pl.loop(0, SPW // LANES)
            def _(i, k=k, tk=tk):
                sl = pl.ds(i * LANES, LANES)
                xa = xe_v[sl]
                inp = lax.shift_right_logical(xa, m16) & one
                isand = lax.shift_right_logical(xa, m17) & one
                d = bit_of(base + i * LANES + iota)
                ready = (1 - d) & (inp | (bit_of(xa & mlow) & bit_of(ye_v[sl])))
                rdy_v[sl] = ready
                lev_v[sl] = jnp.where(ready == 1, tk, lev_v[sl])
                acc_v[3 * k + 0, :] = acc_v[3 * k + 0, :] + ready
                acc_v[3 * k + 1, :] = acc_v[3 * k + 1, :] + (
                    ready & (1 - isand) & (1 - inp))
                acc_v[3 * k + 2, :] = acc_v[3 * k + 2, :] + (ready & isand)

            @pl.loop(0, WPW // LANES)
            def _(wc):
                words = jnp.zeros((LANES,), jnp.int32)
                for b in range(32):
                    bits = plsc.load_gather(rdy_v, [wc * 512 + iota * 32 + b])
                    words = words | lax.shift_left(
                        bits, jnp.full((LANES,), b, jnp.int32))
                old = dbits_v[pl.ds(wbase + wc * LANES, LANES)]
                dnew_v[pl.ds(wc * LANES, LANES)] = old | words

            pltpu.sync_copy(dnew_v, done_out.at[pl.ds(wbase, WPW)])
            if k < KSCH - 1:
                plsc.subcore_barrier()
                pltpu.sync_copy(done_out, dbits_v)
                plsc.subcore_barrier()

        pltpu.sync_copy(lev_v, lev_out.at[pl.ds(base, SPW)])
        pltpu.sync_copy(acc_v, cnt_hbm.at[wid])

    return sched_kernel


@functools.lru_cache(maxsize=None)
def _build_level_kernel(n_pad, e, cmax):
    rpw = cmax // NW

    @functools.partial(
        pl.kernel,
        out_type=(
            jax.ShapeDtypeStruct((cmax, e), jnp.float32),
            jax.ShapeDtypeStruct((cmax, e), jnp.float32),
        ),
        mesh=_sc_mesh(),
        scratch_types=[
            pltpu.VMEM((rpw,), jnp.int32),
            pltpu.VMEM((rpw,), jnp.int32),
            pltpu.VMEM((rpw,), jnp.int32),
            pltpu.VMEM((rpw,), jnp.int32),
            pltpu.VMEM((rpw,), jnp.int32),
            pltpu.VMEM((rpw, e), jnp.float32),
            pltpu.VMEM((rpw, e), jnp.float32),
            pltpu.VMEM((rpw, e), jnp.float32),
            pltpu.SemaphoreType.DMA,
        ],
        name=f"fen_level_{cmax}",
    )
    def level_kernel(idn_hbm, ida_hbm, xe_hbm, ye_hbm, embd_ref,
                     hx_hbm, hy_hbm,
                     idn_v, ida_v, xn_v, xs_v, ys_v, rn_v, hx_v, hy_v, sem):
        base = _wid() * rpw
        pltpu.sync_copy(idn_hbm.at[pl.ds(base, rpw)], idn_v)
        pltpu.sync_copy(ida_hbm.at[pl.ds(base, rpw)], ida_v)
        c1 = pltpu.async_copy(xe_hbm.at[idn_v], xn_v, sem)
        c2 = pltpu.async_copy(xe_hbm.at[ida_v], xs_v, sem)
        c3 = pltpu.async_copy(ye_hbm.at[ida_v], ys_v, sem)
        c1.wait(); c2.wait(); c3.wait()
        c4 = pltpu.async_copy(embd_ref.at[xn_v], rn_v, sem)
        c5 = pltpu.async_copy(embd_ref.at[xs_v], hx_v, sem)
        c6 = pltpu.async_copy(embd_ref.at[ys_v], hy_v, sem)
        c4.wait(); c5.wait(); c6.wait()

        @pl.loop(0, rpw)
        def _(i):
            for j in range(e // LANES):
                sl = (i, pl.ds(j * LANES, LANES))
                rn_v[sl] = -rn_v[sl]

        c7 = pltpu.async_copy(rn_v, embd_ref.at[idn_v], sem)
        pltpu.sync_copy(hx_v, hx_hbm.at[pl.ds(base, rpw)])
        pltpu.sync_copy(hy_v, hy_hbm.at[pl.ds(base, rpw)])
        c7.wait()

    return level_kernel


@functools.lru_cache(maxsize=None)
def _build_level_merged_kernel(n_pad, e, cmax):
    nw1 = LANES
    rpw = cmax // nw1
    mesh = plsc.VectorSubcoreMesh(
        core_axis_name="c", subcore_axis_name="s", num_cores=1)

    @functools.partial(
        pl.kernel,
        out_type=(
            jax.ShapeDtypeStruct((cmax, e), jnp.float32),
            jax.ShapeDtypeStruct((cmax, e), jnp.float32),
        ),
        mesh=mesh,
        scratch_types=[
            pltpu.VMEM((rpw,), jnp.int32),
            pltpu.VMEM((rpw, e), jnp.float32),
            pltpu.VMEM((rpw,), jnp.int32),
            pltpu.VMEM((rpw,), jnp.int32),
            pltpu.VMEM((rpw,), jnp.int32),
            pltpu.VMEM((rpw,), jnp.int32),
            pltpu.VMEM((rpw,), jnp.int32),
            pltpu.VMEM((rpw, e), jnp.float32),
            pltpu.VMEM((rpw, e), jnp.float32),
            pltpu.VMEM((rpw, e), jnp.float32),
            pltpu.SemaphoreType.DMA,
        ],
        name=f"fen_level_m{cmax}",
    )
    def level_merged(pid_hbm, prow_hbm, idn_hbm, ida_hbm, xe_hbm, ye_hbm,
                     embd_ref, hx_hbm, hy_hbm,
                     pid_v, prow_v, idn_v, ida_v, xn_v, xs_v, ys_v,
                     rn_v, hx_v, hy_v, sem):
        base = lax.axis_index("s") * rpw
        a1 = pltpu.async_copy(pid_hbm.at[pl.ds(base, rpw)], pid_v, sem)
        a2 = pltpu.async_copy(prow_hbm.at[pl.ds(base, rpw)], prow_v, sem)
        a3 = pltpu.async_copy(idn_hbm.at[pl.ds(base, rpw)], idn_v, sem)
        a4 = pltpu.async_copy(ida_hbm.at[pl.ds(base, rpw)], ida_v, sem)
        a1.wait(); a2.wait()
        pltpu.async_copy(prow_v, embd_ref.at[pid_v], sem).wait()
        a3.wait(); a4.wait()
        c1 = pltpu.async_copy(xe_hbm.at[idn_v], xn_v, sem)
        c2 = pltpu.async_copy(xe_hbm.at[ida_v], xs_v, sem)
        c3 = pltpu.async_copy(ye_hbm.at[ida_v], ys_v, sem)
        plsc.subcore_barrier()
        c1.wait(); c2.wait(); c3.wait()
        c4 = pltpu.async_copy(embd_ref.at[xn_v], rn_v, sem)
        c5 = pltpu.async_copy(embd_ref.at[xs_v], hx_v, sem)
        c6 = pltpu.async_copy(embd_ref.at[ys_v], hy_v, sem)
        c4.wait(); c5.wait(); c6.wait()

        @pl.loop(0, rpw)
        def _(i):
            for j in range(e // LANES):
                sl = (i, pl.ds(j * LANES, LANES))
                rn_v[sl] = -rn_v[sl]

        c7 = pltpu.async_copy(rn_v, embd_ref.at[idn_v], sem)
        pltpu.sync_copy(hx_v, hx_hbm.at[pl.ds(base, rpw)])
        pltpu.sync_copy(hy_v, hy_hbm.at[pl.ds(base, rpw)])
        c7.wait()

    return level_merged


@functools.lru_cache(maxsize=None)
def _build_scatter_kernel(n_pad, e, cmax):
    rpw = cmax // NW

    @functools.partial(
        pl.kernel,
        out_type=(),
        mesh=_sc_mesh(),
        scratch_types=[
            pltpu.VMEM((rpw,), jnp.int32),
            pltpu.VMEM((rpw, e), jnp.float32),
            pltpu.SemaphoreType.DMA,
        ],
        name=f"fen_scatter_{cmax}",
    )
    def scatter_kernel(ids_hbm, rows_hbm, embd_ref, ids_v, rows_v, sem):
        base = _wid() * rpw
        pltpu.sync_copy(ids_hbm.at[pl.ds(base, rpw)], ids_v)
        pltpu.sync_copy(rows_hbm.at[pl.ds(base, rpw)], rows_v)
        pltpu.async_copy(rows_v, embd_ref.at[ids_v], sem).wait()

    return scatter_kernel


CMAX_S = 768


def _mlp_body(cnt_ref, hx_ref, hy_ref, w0x_ref, w0y_ref, b0_ref, w1_ref,
              b1_ref, g_ref, bb_ref, out_ref):
    t = pl.program_id(0)

    @pl.when(t * TM < cnt_ref[0])
    def _():
        hx = hx_ref[...]
        hy = hy_ref[...]
        z = lax.dot_general(hx, w0x_ref[...], (((1,), (1,)), ((), ())),
                            preferred_element_type=jnp.float32)
        z += lax.dot_general(hy, w0y_ref[...], (((1,), (1,)), ((), ())),
                             preferred_element_type=jnp.float32)
        z = jnp.maximum(z + b0_ref[...], 0.0)
        o = lax.dot_general(z, w1_ref[...], (((1,), (1,)), ((), ())),
                            preferred_element_type=jnp.float32)
        o = o + b1_ref[...]
        mu = jnp.mean(o, axis=-1, keepdims=True)
        var = jnp.mean((o - mu) ** 2, axis=-1, keepdims=True)
        out_ref[...] = (o - mu) * lax.rsqrt(var + 1e-5) * g_ref[...] + bb_ref[...]


@functools.lru_cache(maxsize=None)
def _build_mlp_kernel(e, h, cmax):
    grid = (cmax // TM,)
    return pl.pallas_call(
        _mlp_body,
        grid_spec=pltpu.PrefetchScalarGridSpec(
            num_scalar_prefetch=1,
            grid=grid,
            in_specs=[
                pl.BlockSpec((TM, e), lambda t, cnt: (t, 0)),
                pl.BlockSpec((TM, e), lambda t, cnt: (t, 0)),
                pl.BlockSpec((h, e), lambda t, cnt: (0, 0)),
                pl.BlockSpec((h, e), lambda t, cnt: (0, 0)),
                pl.BlockSpec((1, h), lambda t, cnt: (0, 0)),
                pl.BlockSpec((e, h), lambda t, cnt: (0, 0)),
                pl.BlockSpec((1, e), lambda t, cnt: (0, 0)),
                pl.BlockSpec((1, e), lambda t, cnt: (0, 0)),
                pl.BlockSpec((1, e), lambda t, cnt: (0, 0)),
            ],
            out_specs=pl.BlockSpec((TM, e), lambda t, cnt: (t, 0)),
        ),
        out_shape=jax.ShapeDtypeStruct((cmax, e), jnp.float32),
    )


def kernel(emb, W0, b0, W1, b1, ln_g, ln_b, nodes, x_edges, y_edges):
    n, e = emb.shape
    hdim = W0.shape[0]
    n_pad = n + CMAX
    is_input = nodes == 0
    n_inputs = jnp.sum(is_input)

    init = jnp.where(jnp.arange(n)[:, None] < n_inputs, emb,
                     jnp.zeros((n, e), emb.dtype))
    embd_ext = jnp.concatenate(
        [init, jnp.zeros((n_pad - n, e), emb.dtype)], axis=0)
    dummy_tail = jnp.arange(n, n_pad, dtype=jnp.int32)
    xe_ext = jnp.concatenate([x_edges.astype(jnp.int32), dummy_tail])
    ye_ext = jnp.concatenate([y_edges.astype(jnp.int32), dummy_tail])

    big = jnp.int32(0x3FFFFFFF)
    pad_sched = N_SCHED - n
    xe_sched = jnp.concatenate(
        [x_edges.astype(jnp.int32)
         | (is_input.astype(jnp.int32) << 16)
         | ((nodes == 1).astype(jnp.int32) << 17),
         jnp.full((pad_sched,), n, jnp.int32)])
    ye_sched = jnp.concatenate(
        [y_edges.astype(jnp.int32), jnp.full((pad_sched,), n, jnp.int32)])
    sched_k = _build_sched_kernel()

    def sched_cond(state):
        t, cnt, _, _, _ = state
        return cnt > 0

    def sched_body(state):
        t, _, done, lev, cnts = state
        t_arr = jnp.full((LANES,), t, jnp.int32)
        done, lev, counts = sched_k(t_arr, xe_sched, ye_sched, done, lev)
        rows = jnp.sum(counts, axis=(0, 2)).reshape(KSCH, 3)
        cnts = lax.dynamic_update_slice(
            cnts, rows, (jnp.minimum(t, MAXD - KSCH), 0))
        return t + KSCH, rows[KSCH - 1, 0], done, lev, cnts

    state0 = (jnp.int32(0), jnp.int32(1), jnp.zeros((NWRD,), jnp.int32),
              jnp.full((N_SCHED,), big, jnp.int32),
              jnp.zeros((MAXD, 3), jnp.int32))
    state0 = lax.fori_loop(0, 6, lambda i, s: sched_body(s), state0)
    _, _, _, lev_full, cnts = lax.while_loop(
        sched_cond, sched_body, state0)
    lev = lev_full[:n]
    depth_levels = jnp.sum((cnts[:, 0] > 0).astype(jnp.int32))

    key = jnp.where((lev > 0) & (lev < big),
                    lev * 2 + (nodes == 1).astype(jnp.int32),
                    jnp.int32(2 * MAXD + 2))
    key = jnp.minimum(key, 2 * MAXD + 2)
    packed = (key << 16) | jnp.arange(n, dtype=jnp.int32)
    order = lax.sort(packed) & jnp.int32(0xFFFF)
    offs = jnp.concatenate(
        [jnp.zeros((1,), jnp.int32), jnp.cumsum(cnts[:, 1:3].reshape(-1))])
    order_pad = jnp.concatenate(
        [order, jnp.full((CMAX,), n, dtype=jnp.int32)])

    level_kb = _build_level_kernel(n_pad, e, CMAX)
    scatter_kb = _build_scatter_kernel(n_pad, e, CMAX)
    mlp_kb = _build_mlp_kernel(e, hdim, CMAX)
    merged_ks = _build_level_merged_kernel(n_pad, e, CMAX_S)
    scatter_ks = _build_scatter_kernel(n_pad, e, CMAX_S)
    mlp_ks = _build_mlp_kernel(e, hdim, CMAX_S)

    w0x = W0[:, :e]
    w0y = W0[:, e:]
    b0r = b0.reshape(1, hdim)
    b1r = b1.reshape(1, e)
    gr = ln_g.reshape(1, e)
    br = ln_b.reshape(1, e)

    embd_ref = jax.new_ref(embd_ext)

    def make_ids(l, cmax):
        slot = jnp.arange(cmax, dtype=jnp.int32)
        dummy_ids = slot + n
        s0 = offs[2 * l]
        s1 = offs[2 * l + 1]
        s2 = offs[2 * l + 2]
        ids_not = lax.dynamic_slice(order_pad, (s0,), (cmax,))
        ids_not = jnp.where(slot < s1 - s0, ids_not, dummy_ids)
        cnt_and = s2 - s1
        ids_and = lax.dynamic_slice(order_pad, (s1,), (cmax,))
        ids_and = jnp.where(slot < cnt_and, ids_and, dummy_ids)
        return ids_not, ids_and, cnt_and

    def level_big(l):
        ids_not, ids_and, cnt_and = make_ids(l, CMAX)
        hx, hy = level_kb(ids_not, ids_and, xe_ext, ye_ext, embd_ref)
        out = mlp_kb(cnt_and.reshape(1), hx, hy, w0x, w0y, b0r, W1, b1r,
                     gr, br)
        scatter_kb(ids_and, out, embd_ref)

    level_big(jnp.int32(1))
    level_big(jnp.int32(2))

    def level_body(l, carry):
        pids, pout = carry
        ids_not, ids_and, cnt_and = make_ids(l, CMAX_S)
        hx, hy = merged_ks(pids, pout, ids_not, ids_and, xe_ext, ye_ext,
                           embd_ref)
        out = mlp_ks(cnt_and.reshape(1), hx, hy, w0x, w0y, b0r, W1, b1r,
                     gr, br)
        return ids_and, out

    dummy_s = jnp.arange(CMAX_S, dtype=jnp.int32) + n
    pids, pout = lax.fori_loop(
        3, jnp.minimum(depth_levels, MAXD), level_body,
        (dummy_s, jnp.zeros((CMAX_S, e), jnp.float32)))
    scatter_ks(pids, pout, embd_ref)
    return embd_ref[...][:n]


# speedup vs baseline: 1.0698x; 1.0535x over previous
import functools

import jax
import jax.numpy as jnp
from jax import lax
from jax.experimental import pallas as pl
from jax.experimental.pallas import tpu as pltpu
from jax.experimental.pallas import tpu_sc as plsc

MAXD = 512
CMAX = 3072
TM = 256
NW = 32
RPW = CMAX // NW
LANES = 16


def _sc_mesh():
    return plsc.VectorSubcoreMesh(core_axis_name="c", subcore_axis_name="s")


def _wid():
    return lax.axis_index("s") * 2 + lax.axis_index("c")


N_SCHED = 65536
NWRD = N_SCHED // 32
KSCH = 3
NW1 = LANES
SPW = N_SCHED // NW1
WPW = NWRD // NW1


@functools.lru_cache(maxsize=None)
def _build_sched_kernel():
    @functools.partial(
        pl.kernel,
        out_type=(
            jax.ShapeDtypeStruct((NWRD,), jnp.int32),
            jax.ShapeDtypeStruct((N_SCHED,), jnp.int32),
            jax.ShapeDtypeStruct((NW1, KSCH * 3, LANES), jnp.int32),
        ),
        mesh=plsc.VectorSubcoreMesh(
            core_axis_name="c", subcore_axis_name="s", num_cores=1),
        scratch_types=[
            pltpu.VMEM((NWRD,), jnp.int32),
            pltpu.VMEM((SPW,), jnp.int32),
            pltpu.VMEM((SPW,), jnp.int32),
            pltpu.VMEM((SPW,), jnp.int32),
            pltpu.VMEM((SPW,), jnp.int32),
            pltpu.VMEM((WPW,), jnp.int32),
            pltpu.VMEM((LANES,), jnp.int32),
            pltpu.VMEM((KSCH * 3, LANES), jnp.int32),
            pltpu.SemaphoreType.DMA,
        ],
        compiler_params=pltpu.CompilerParams(needs_layout_passes=False),
        name="fen_sched",
    )
    def sched_kernel(t_hbm, xe_hbm, ye_hbm, done_hbm,
                     lev_hbm, done_out, lev_out, cnt_hbm,
                     dbits_v, xe_v, ye_v, lev_v, rdy_v,
                     dnew_v, t_v, acc_v, sem):
        wid = lax.axis_index("s")
        base = wid * SPW
        wbase = wid * WPW
        c0 = pltpu.async_copy(done_hbm, dbits_v, sem)
        c1 = pltpu.async_copy(xe_hbm.at[pl.ds(base, SPW)], xe_v, sem)
        c2 = pltpu.async_copy(ye_hbm.at[pl.ds(base, SPW)], ye_v, sem)
        c3 = pltpu.async_copy(lev_hbm.at[pl.ds(base, SPW)], lev_v, sem)
        c4 = pltpu.async_copy(t_hbm, t_v, sem)
        c0.wait(); c1.wait(); c2.wait(); c3.wait(); c4.wait()
        iota = lax.iota(jnp.int32, LANES)
        one = jnp.ones((LANES,), jnp.int32)
        five = jnp.full((LANES,), 5, jnp.int32)
        m31 = jnp.full((LANES,), 31, jnp.int32)
        m16 = jnp.full((LANES,), 16, jnp.int32)
        m17 = jnp.full((LANES,), 17, jnp.int32)
        mlow = jnp.full((LANES,), 0xFFFF, jnp.int32)

        def bit_of(idx):
            w = plsc.load_gather(dbits_v, [lax.shift_right_logical(idx, five)])
            return lax.shift_right_logical(w, idx & m31) & one

        for k in range(KSCH):
            acc_v[3 * k + 0, :] = jnp.zeros((LANES,), jnp.int32)
            acc_v[3 * k + 1, :] = jnp.zeros((LANES,), jnp.int32)
            acc_v[3 * k + 2, :] = jnp.zeros((LANES,), jnp.int32)
            tk = t_v[...] + k

            @pl.loop(0, SPW // LANES)
            def _(i, k=k, tk=tk):
                sl = pl.ds(i * LANES, LANES)
                xa = xe_v[sl]
                inp = lax.shift_right_logical(xa, m16) & one
                isand = lax.shift_right_logical(xa, m17) & one
                d = bit_of(base + i * LANES + iota)
                ready = (1 - d) & (inp | (bit_of(xa & mlow) & bit_of(ye_v[sl])))
                rdy_v[sl] = ready
                lev_v[sl] = jnp.where(ready == 1, tk, lev_v[sl])
                acc_v[3 * k + 0, :] = acc_v[3 * k + 0, :] + ready
                acc_v[3 * k + 1, :] = acc_v[3 * k + 1, :] + (
                    ready & (1 - isand) & (1 - inp))
                acc_v[3 * k + 2, :] = acc_v[3 * k + 2, :] + (ready & isand)

            @pl.loop(0, WPW // LANES)
            def _(wc):
                words = jnp.zeros((LANES,), jnp.int32)
                for b in range(32):
                    bits = plsc.load_gather(rdy_v, [wc * 512 + iota * 32 + b])
                    words = words | lax.shift_left(
                        bits, jnp.full((LANES,), b, jnp.int32))
                old = dbits_v[pl.ds(wbase + wc * LANES, LANES)]
                dnew_v[pl.ds(wc * LANES, LANES)] = old | words

            pltpu.sync_copy(dnew_v, done_out.at[pl.ds(wbase, WPW)])
            if k < KSCH - 1:
                plsc.subcore_barrier()
                pltpu.sync_copy(done_out, dbits_v)
                plsc.subcore_barrier()

        pltpu.sync_copy(lev_v, lev_out.at[pl.ds(base, SPW)])
        pltpu.sync_copy(acc_v, cnt_hbm.at[wid])

    return sched_kernel


@functools.lru_cache(maxsize=None)
def _build_level_kernel(n_pad, e, cmax):
    rpw = cmax // NW

    @functools.partial(
        pl.kernel,
        out_type=(
            jax.ShapeDtypeStruct((cmax, e), jnp.float32),
            jax.ShapeDtypeStruct((cmax, e), jnp.float32),
        ),
        mesh=_sc_mesh(),
        scratch_types=[
            pltpu.VMEM((rpw,), jnp.int32),
            pltpu.VMEM((rpw,), jnp.int32),
            pltpu.VMEM((rpw,), jnp.int32),
            pltpu.VMEM((rpw,), jnp.int32),
            pltpu.VMEM((rpw,), jnp.int32),
            pltpu.VMEM((rpw, e), jnp.float32),
            pltpu.VMEM((rpw, e), jnp.float32),
            pltpu.VMEM((rpw, e), jnp.float32),
            pltpu.SemaphoreType.DMA,
        ],
        name=f"fen_level_{cmax}",
    )
    def level_kernel(idn_hbm, ida_hbm, xe_hbm, ye_hbm, embd_ref,
                     hx_hbm, hy_hbm,
                     idn_v, ida_v, xn_v, xs_v, ys_v, rn_v, hx_v, hy_v, sem):
        base = _wid() * rpw
        pltpu.sync_copy(idn_hbm.at[pl.ds(base, rpw)], idn_v)
        pltpu.sync_copy(ida_hbm.at[pl.ds(base, rpw)], ida_v)
        c1 = pltpu.async_copy(xe_hbm.at[idn_v], xn_v, sem)
        c2 = pltpu.async_copy(xe_hbm.at[ida_v], xs_v, sem)
        c3 = pltpu.async_copy(ye_hbm.at[ida_v], ys_v, sem)
        c1.wait(); c2.wait(); c3.wait()
        c4 = pltpu.async_copy(embd_ref.at[xn_v], rn_v, sem)
        c5 = pltpu.async_copy(embd_ref.at[xs_v], hx_v, sem)
        c6 = pltpu.async_copy(embd_ref.at[ys_v], hy_v, sem)
        c4.wait(); c5.wait(); c6.wait()

        @pl.loop(0, rpw)
        def _(i):
            for j in range(e // LANES):
                sl = (i, pl.ds(j * LANES, LANES))
                rn_v[sl] = -rn_v[sl]

        c7 = pltpu.async_copy(rn_v, embd_ref.at[idn_v], sem)
        pltpu.sync_copy(hx_v, hx_hbm.at[pl.ds(base, rpw)])
        pltpu.sync_copy(hy_v, hy_hbm.at[pl.ds(base, rpw)])
        c7.wait()

    return level_kernel


@functools.lru_cache(maxsize=None)
def _build_level_merged_kernel(n_pad, e, prev_cmax, cmax):
    nw1 = LANES
    rpw = cmax // nw1
    rpw_p = prev_cmax // nw1
    mesh = plsc.VectorSubcoreMesh(
        core_axis_name="c", subcore_axis_name="s", num_cores=1)

    @functools.partial(
        pl.kernel,
        out_type=(
            jax.ShapeDtypeStruct((cmax, e), jnp.float32),
            jax.ShapeDtypeStruct((cmax, e), jnp.float32),
        ),
        mesh=mesh,
        scratch_types=[
            pltpu.VMEM((rpw_p,), jnp.int32),
            pltpu.VMEM((rpw_p, e), jnp.float32),
            pltpu.VMEM((rpw,), jnp.int32),
            pltpu.VMEM((rpw,), jnp.int32),
            pltpu.VMEM((rpw,), jnp.int32),
            pltpu.VMEM((rpw,), jnp.int32),
            pltpu.VMEM((rpw,), jnp.int32),
            pltpu.VMEM((rpw, e), jnp.float32),
            pltpu.VMEM((rpw, e), jnp.float32),
            pltpu.VMEM((rpw, e), jnp.float32),
            pltpu.SemaphoreType.DMA,
        ],
        name=f"fen_level_m{prev_cmax}_{cmax}",
    )
    def level_merged(pid_hbm, prow_hbm, idn_hbm, ida_hbm, xe_hbm, ye_hbm,
                     embd_ref, hx_hbm, hy_hbm,
                     pid_v, prow_v, idn_v, ida_v, xn_v, xs_v, ys_v,
                     rn_v, hx_v, hy_v, sem):
        sid = lax.axis_index("s")
        base = sid * rpw
        base_p = sid * rpw_p
        a1 = pltpu.async_copy(pid_hbm.at[pl.ds(base_p, rpw_p)], pid_v, sem)
        a2 = pltpu.async_copy(prow_hbm.at[pl.ds(base_p, rpw_p)], prow_v, sem)
        a3 = pltpu.async_copy(idn_hbm.at[pl.ds(base, rpw)], idn_v, sem)
        a4 = pltpu.async_copy(ida_hbm.at[pl.ds(base, rpw)], ida_v, sem)
        a1.wait(); a2.wait()
        pltpu.async_copy(prow_v, embd_ref.at[pid_v], sem).wait()
        a3.wait(); a4.wait()
        c1 = pltpu.async_copy(xe_hbm.at[idn_v], xn_v, sem)
        c2 = pltpu.async_copy(xe_hbm.at[ida_v], xs_v, sem)
        c3 = pltpu.async_copy(ye_hbm.at[ida_v], ys_v, sem)
        plsc.subcore_barrier()
        c1.wait(); c2.wait(); c3.wait()
        c4 = pltpu.async_copy(embd_ref.at[xn_v], rn_v, sem)
        c5 = pltpu.async_copy(embd_ref.at[xs_v], hx_v, sem)
        c6 = pltpu.async_copy(embd_ref.at[ys_v], hy_v, sem)
        c4.wait(); c5.wait(); c6.wait()

        @pl.loop(0, rpw)
        def _(i):
            for j in range(e // LANES):
                sl = (i, pl.ds(j * LANES, LANES))
                rn_v[sl] = -rn_v[sl]

        c7 = pltpu.async_copy(rn_v, embd_ref.at[idn_v], sem)
        pltpu.sync_copy(hx_v, hx_hbm.at[pl.ds(base, rpw)])
        pltpu.sync_copy(hy_v, hy_hbm.at[pl.ds(base, rpw)])
        c7.wait()

    return level_merged


@functools.lru_cache(maxsize=None)
def _build_scatter_kernel(n_pad, e, cmax):
    rpw = cmax // NW

    @functools.partial(
        pl.kernel,
        out_type=(),
        mesh=_sc_mesh(),
        scratch_types=[
            pltpu.VMEM((rpw,), jnp.int32),
            pltpu.VMEM((rpw, e), jnp.float32),
            pltpu.SemaphoreType.DMA,
        ],
        name=f"fen_scatter_{cmax}",
    )
    def scatter_kernel(ids_hbm, rows_hbm, embd_ref, ids_v, rows_v, sem):
        base = _wid() * rpw
        pltpu.sync_copy(ids_hbm.at[pl.ds(base, rpw)], ids_v)
        pltpu.sync_copy(rows_hbm.at[pl.ds(base, rpw)], rows_v)
        pltpu.async_copy(rows_v, embd_ref.at[ids_v], sem).wait()

    return scatter_kernel


CMAX_S = 768
CMAX_T = 512


def _mlp_body(cnt_ref, hx_ref, hy_ref, w0x_ref, w0y_ref, b0_ref, w1_ref,
              b1_ref, g_ref, bb_ref, out_ref):
    t = pl.program_id(0)

    @pl.when(t * TM < cnt_ref[0])
    def _():
        hx = hx_ref[...]
        hy = hy_ref[...]
        z = lax.dot_general(hx, w0x_ref[...], (((1,), (1,)), ((), ())),
                            preferred_element_type=jnp.float32)
        z += lax.dot_general(hy, w0y_ref[...], (((1,), (1,)), ((), ())),
                             preferred_element_type=jnp.float32)
        z = jnp.maximum(z + b0_ref[...], 0.0)
        o = lax.dot_general(z, w1_ref[...], (((1,), (1,)), ((), ())),
                            preferred_element_type=jnp.float32)
        o = o + b1_ref[...]
        mu = jnp.mean(o, axis=-1, keepdims=True)
        var = jnp.mean((o - mu) ** 2, axis=-1, keepdims=True)
        out_ref[...] = (o - mu) * lax.rsqrt(var + 1e-5) * g_ref[...] + bb_ref[...]


@functools.lru_cache(maxsize=None)
def _build_mlp_kernel(e, h, cmax):
    grid = (cmax // TM,)
    return pl.pallas_call(
        _mlp_body,
        grid_spec=pltpu.PrefetchScalarGridSpec(
            num_scalar_prefetch=1,
            grid=grid,
            in_specs=[
                pl.BlockSpec((TM, e), lambda t, cnt: (t, 0)),
                pl.BlockSpec((TM, e), lambda t, cnt: (t, 0)),
                pl.BlockSpec((h, e), lambda t, cnt: (0, 0)),
                pl.BlockSpec((h, e), lambda t, cnt: (0, 0)),
                pl.BlockSpec((1, h), lambda t, cnt: (0, 0)),
                pl.BlockSpec((e, h), lambda t, cnt: (0, 0)),
                pl.BlockSpec((1, e), lambda t, cnt: (0, 0)),
                pl.BlockSpec((1, e), lambda t, cnt: (0, 0)),
                pl.BlockSpec((1, e), lambda t, cnt: (0, 0)),
            ],
            out_specs=pl.BlockSpec((TM, e), lambda t, cnt: (t, 0)),
        ),
        out_shape=jax.ShapeDtypeStruct((cmax, e), jnp.float32),
    )


def kernel(emb, W0, b0, W1, b1, ln_g, ln_b, nodes, x_edges, y_edges):
    n, e = emb.shape
    hdim = W0.shape[0]
    n_pad = n + CMAX
    is_input = nodes == 0
    n_inputs = jnp.sum(is_input)

    init = jnp.where(jnp.arange(n)[:, None] < n_inputs, emb,
                     jnp.zeros((n, e), emb.dtype))
    embd_ext = jnp.concatenate(
        [init, jnp.zeros((n_pad - n, e), emb.dtype)], axis=0)
    dummy_tail = jnp.arange(n, n_pad, dtype=jnp.int32)
    xe_ext = jnp.concatenate([x_edges.astype(jnp.int32), dummy_tail])
    ye_ext = jnp.concatenate([y_edges.astype(jnp.int32), dummy_tail])

    big = jnp.int32(0x3FFFFFFF)
    pad_sched = N_SCHED - n
    xe_sched = jnp.concatenate(
        [x_edges.astype(jnp.int32)
         | (is_input.astype(jnp.int32) << 16)
         | ((nodes == 1).astype(jnp.int32) << 17),
         jnp.full((pad_sched,), n, jnp.int32)])
    ye_sched = jnp.concatenate(
        [y_edges.astype(jnp.int32), jnp.full((pad_sched,), n, jnp.int32)])
    sched_k = _build_sched_kernel()

    def sched_cond(state):
        t, cnt, _, _, _ = state
        return cnt > 0

    def sched_body(state):
        t, _, done, lev, cnts = state
        t_arr = jnp.full((LANES,), t, jnp.int32)
        done, lev, counts = sched_k(t_arr, xe_sched, ye_sched, done, lev)
        rows = jnp.sum(counts, axis=(0, 2)).reshape(KSCH, 3)
        cnts = lax.dynamic_update_slice(
            cnts, rows, (jnp.minimum(t, MAXD - KSCH), 0))
        return t + KSCH, rows[KSCH - 1, 0], done, lev, cnts

    state0 = (jnp.int32(0), jnp.int32(1), jnp.zeros((NWRD,), jnp.int32),
              jnp.full((N_SCHED,), big, jnp.int32),
              jnp.zeros((MAXD, 3), jnp.int32))
    state0 = lax.fori_loop(0, 6, lambda i, s: sched_body(s), state0)
    _, _, _, lev_full, cnts = lax.while_loop(
        sched_cond, sched_body, state0)
    lev = lev_full[:n]
    depth_levels = jnp.sum((cnts[:, 0] > 0).astype(jnp.int32))

    key = jnp.where((lev > 0) & (lev < big),
                    lev * 2 + (nodes == 1).astype(jnp.int32),
                    jnp.int32(2 * MAXD + 2))
    key = jnp.minimum(key, 2 * MAXD + 2)
    packed = (key << 16) | jnp.arange(n, dtype=jnp.int32)
    order = lax.sort(packed) & jnp.int32(0xFFFF)
    offs = jnp.concatenate(
        [jnp.zeros((1,), jnp.int32), jnp.cumsum(cnts[:, 1:3].reshape(-1))])
    order_pad = jnp.concatenate(
        [order, jnp.full((CMAX,), n, dtype=jnp.int32)])

    level_kb = _build_level_kernel(n_pad, e, CMAX)
    scatter_kb = _build_scatter_kernel(n_pad, e, CMAX)
    mlp_kb = _build_mlp_kernel(e, hdim, CMAX)
    merged_77 = _build_level_merged_kernel(n_pad, e, CMAX_S, CMAX_S)
    merged_75 = _build_level_merged_kernel(n_pad, e, CMAX_S, CMAX_T)
    merged_55 = _build_level_merged_kernel(n_pad, e, CMAX_T, CMAX_T)
    scatter_kt = _build_scatter_kernel(n_pad, e, CMAX_T)
    mlp_ks = _build_mlp_kernel(e, hdim, CMAX_S)
    mlp_kt = _build_mlp_kernel(e, hdim, CMAX_T)

    w0x = W0[:, :e]
    w0y = W0[:, e:]
    b0r = b0.reshape(1, hdim)
    b1r = b1.reshape(1, e)
    gr = ln_g.reshape(1, e)
    br = ln_b.reshape(1, e)

    embd_ref = jax.new_ref(embd_ext)

    def make_ids(l, cmax):
        slot = jnp.arange(cmax, dtype=jnp.int32)
        dummy_ids = slot + n
        s0 = offs[2 * l]
        s1 = offs[2 * l + 1]
        s2 = offs[2 * l + 2]
        ids_not = lax.dynamic_slice(order_pad, (s0,), (cmax,))
        ids_not = jnp.where(slot < s1 - s0, ids_not, dummy_ids)
        cnt_and = s2 - s1
        ids_and = lax.dynamic_slice(order_pad, (s1,), (cmax,))
        ids_and = jnp.where(slot < cnt_and, ids_and, dummy_ids)
        return ids_not, ids_and, cnt_and

    def level_big(l):
        ids_not, ids_and, cnt_and = make_ids(l, CMAX)
        hx, hy = level_kb(ids_not, ids_and, xe_ext, ye_ext, embd_ref)
        out = mlp_kb(cnt_and.reshape(1), hx, hy, w0x, w0y, b0r, W1, b1r,
                     gr, br)
        scatter_kb(ids_and, out, embd_ref)

    level_big(jnp.int32(1))
    level_big(jnp.int32(2))

    def run_level(merged_k, mlp_k, cmax, l, pids, pout):
        ids_not, ids_and, cnt_and = make_ids(l, cmax)
        hx, hy = merged_k(pids, pout, ids_not, ids_and, xe_ext, ye_ext,
                          embd_ref)
        out = mlp_k(cnt_and.reshape(1), hx, hy, w0x, w0y, b0r, W1, b1r,
                    gr, br)
        return ids_and, out

    dummy_s = jnp.arange(CMAX_S, dtype=jnp.int32) + n
    pids, pout = dummy_s, jnp.zeros((CMAX_S, e), jnp.float32)
    pids, pout = run_level(merged_77, mlp_ks, CMAX_S, jnp.int32(3),
                           pids, pout)
    pids, pout = run_level(merged_77, mlp_ks, CMAX_S, jnp.int32(4),
                           pids, pout)
    pids, pout = run_level(merged_75, mlp_kt, CMAX_T, jnp.int32(5),
                           pids, pout)

    def level_body(l, carry):
        pids, pout = carry
        return run_level(merged_55, mlp_kt, CMAX_T, l, pids, pout)

    pids, pout = lax.fori_loop(
        6, jnp.minimum(depth_levels, MAXD), level_body, (pids, pout))
    scatter_kt(pids, pout, embd_ref)
    return embd_ref[...][:n]
